# baseline (speedup 1.0000x reference)
import functools

import jax
import jax.numpy as jnp
from jax import lax
from jax.experimental import pallas as pl
from jax.experimental.pallas import tpu as pltpu
from jax.experimental.pallas import tpu_sc as plsc

E = 8
TOP_K = 2
LANES = 128
BMR = 256
BM = 256
NW = 32


def _router_body(h_ref, gw_ref, ids_ref, gates_ref, soft_ref, cnt_ref, aux_ref):
    i = pl.program_id(0)
    nsteps = pl.num_programs(0)
    x = h_ref[...].astype(jnp.bfloat16)
    gw = gw_ref[...].astype(jnp.bfloat16)
    logits = jnp.dot(x, gw, preferred_element_type=jnp.float32)
    lane = lax.broadcasted_iota(jnp.int32, logits.shape, 1)
    neg = jnp.float32(-jnp.inf)
    l = jnp.where(lane < E, logits, neg)
    big = jnp.int32(10**9)
    m1 = jnp.max(l, axis=1, keepdims=True)
    i1 = jnp.min(jnp.where(l == m1, lane, big), axis=1, keepdims=True)
    l2 = jnp.where(lane == i1, neg, l)
    m2 = jnp.max(l2, axis=1, keepdims=True)
    i2 = jnp.min(jnp.where(l2 == m2, lane, big), axis=1, keepdims=True)
    ew = jnp.exp(m2 - m1)
    d = 1.0 + ew
    ids_ref[0, 0, :] = i1[:, 0]
    ids_ref[0, 1, :] = i2[:, 0]
    gates_ref[0, 0, :] = (1.0 / d)[:, 0]
    gates_ref[0, 1, :] = (ew / d)[:, 0]
    p = jnp.exp(l - m1)
    p = p / jnp.sum(p, axis=1, keepdims=True)
    soft_part = jnp.sum(p, axis=0, keepdims=True)
    oh = (lane == i1).astype(jnp.float32) + (lane == i2).astype(jnp.float32)
    cnt_part = jnp.sum(oh, axis=0, keepdims=True)

    @pl.when(i == 0)
    def _():
        soft_ref[...] = jnp.zeros_like(soft_ref)
        cnt_ref[...] = jnp.zeros_like(cnt_ref)

    soft_ref[...] += soft_part
    cnt_ref[...] += cnt_part

    @pl.when(i == nsteps - 1)
    def _():
        t_tot = jnp.float32(nsteps * BMR)
        total = jnp.sum(soft_ref[...] * cnt_ref[...]) * (E / (t_tot * t_tot))
        aux_ref[...] = jnp.reshape(total, (1, 1))


def _router(h2d, gate_w):
    T, H = h2d.shape
    nb = T // BMR
    gw_pad = jnp.pad(gate_w, ((0, 0), (0, LANES - E)))
    return pl.pallas_call(
        _router_body,
        grid=(nb,),
        in_specs=[
            pl.BlockSpec((BMR, H), lambda i: (i, 0)),
            pl.BlockSpec((H, LANES), lambda i: (0, 0)),
        ],
        out_specs=[
            pl.BlockSpec((1, 2, BMR), lambda i: (i, 0, 0)),
            pl.BlockSpec((1, 2, BMR), lambda i: (i, 0, 0)),
            pl.BlockSpec((1, LANES), lambda i: (0, 0)),
            pl.BlockSpec((1, LANES), lambda i: (0, 0)),
            pl.BlockSpec((1, 1), lambda i: (0, 0)),
        ],
        out_shape=[
            jax.ShapeDtypeStruct((nb, 2, BMR), jnp.int32),
            jax.ShapeDtypeStruct((nb, 2, BMR), jnp.float32),
            jax.ShapeDtypeStruct((1, LANES), jnp.float32),
            jax.ShapeDtypeStruct((1, LANES), jnp.float32),
            jax.ShapeDtypeStruct((1, 1), jnp.float32),
        ],
    )(h2d, gw_pad)


def _make_gather(T, H, NP):
    rows_per_w = NP // NW
    CH = 96
    nch = rows_per_w // CH
    mesh = plsc.VectorSubcoreMesh(core_axis_name="c", subcore_axis_name="s")

    @functools.partial(
        pl.kernel,
        out_type=jax.ShapeDtypeStruct((NP, H), jnp.float32),
        mesh=mesh,
        scratch_types=[
            pltpu.VMEM((CH,), jnp.int32),
            pltpu.VMEM((CH, H), jnp.float32),
            pltpu.SemaphoreType.DMA,
        ],
    )
    def gather(h_hbm, tid_hbm, out_hbm, idx_v, rows_v, sem):
        wid = lax.axis_index("s") * 2 + lax.axis_index("c")
        base = wid * rows_per_w
        for ch in range(nch):
            off = base + ch * CH
            pltpu.sync_copy(tid_hbm.at[pl.ds(off, CH)], idx_v)
            pltpu.async_copy(h_hbm.at[idx_v], rows_v, sem).wait()
            pltpu.sync_copy(rows_v, out_hbm.at[pl.ds(off, CH)])

    return gather


def _gemm_body(xi_ref, wi_ref, vd_ref, x_ref, g_ref, w1_ref, w3_ref, w2_ref, y_ref):
    b = pl.program_id(0)

    @pl.when(vd_ref[b] == 1)
    def _():
        x = x_ref[...]
        a = jnp.dot(x, w1_ref[0], preferred_element_type=jnp.float32)
        c = jnp.dot(x, w3_ref[0], preferred_element_type=jnp.float32)
        inner = (a * jax.nn.sigmoid(a)) * c
        g = g_ref[0, 0, :]
        inner = inner * g[:, None]
        y_ref[...] = jnp.dot(
            inner.astype(jnp.bfloat16), w2_ref[0], preferred_element_type=jnp.float32
        )


def _gemm(xb, g3d, w1b, w3b, w2b, xi, wi, vd, NP, H, F):
    nblk = NP // BM
    grid_spec = pltpu.PrefetchScalarGridSpec(
        num_scalar_prefetch=3,
        grid=(nblk,),
        in_specs=[
            pl.BlockSpec((BM, H), lambda b, xi, wi, vd: (xi[b], 0)),
            pl.BlockSpec((1, 1, BM), lambda b, xi, wi, vd: (xi[b], 0, 0)),
            pl.BlockSpec((1, H, F), lambda b, xi, wi, vd: (wi[b], 0, 0)),
            pl.BlockSpec((1, H, F), lambda b, xi, wi, vd: (wi[b], 0, 0)),
            pl.BlockSpec((1, F, H), lambda b, xi, wi, vd: (wi[b], 0, 0)),
        ],
        out_specs=pl.BlockSpec((BM, H), lambda b, xi, wi, vd: (xi[b], 0)),
    )
    return pl.pallas_call(
        _gemm_body,
        grid_spec=grid_spec,
        out_shape=jax.ShapeDtypeStruct((NP, H), jnp.float32),
    )(xi, wi, vd, xb, g3d, w1b, w3b, w2b)


def _make_combine(T, H, NP):
    tpw = T // NW
    CT = 32
    nch = tpw // CT
    nc16 = H // 16
    mesh = plsc.VectorSubcoreMesh(core_axis_name="c", subcore_axis_name="s")

    @functools.partial(
        pl.kernel,
        out_type=jax.ShapeDtypeStruct((T, H), jnp.float32),
        mesh=mesh,
        scratch_types=[
            pltpu.VMEM((CT,), jnp.int32),
            pltpu.VMEM((CT,), jnp.int32),
            pltpu.VMEM((CT, H), jnp.float32),
            pltpu.VMEM((CT, H), jnp.float32),
            pltpu.SemaphoreType.DMA,
            pltpu.SemaphoreType.DMA,
        ],
    )
    def combine(y_hbm, p0_hbm, p1_hbm, out_hbm, ia_v, ib_v, ra_v, rb_v, sa, sb):
        wid = lax.axis_index("s") * 2 + lax.axis_index("c")
        base = wid * tpw
        for ch in range(nch):
            off = base + ch * CT
            pltpu.sync_copy(p0_hbm.at[pl.ds(off, CT)], ia_v)
            pltpu.sync_copy(p1_hbm.at[pl.ds(off, CT)], ib_v)
            ca = pltpu.async_copy(y_hbm.at[ia_v], ra_v, sa)
            cb = pltpu.async_copy(y_hbm.at[ib_v], rb_v, sb)
            ca.wait()
            cb.wait()

            def body(t, carry):
                for cc in range(nc16):
                    sl = pl.ds(cc * 16, 16)
                    ra_v[t, sl] = ra_v[t, sl] + rb_v[t, sl]
                return carry

            lax.fori_loop(0, CT, body, 0)
            pltpu.sync_copy(ra_v, out_hbm.at[pl.ds(off, CT)])

    return combine


def kernel(hidden_states, gate_w, w1, w3, w2):
    B, S, H = hidden_states.shape
    F = w1.shape[-1]
    T = B * S
    P = T * TOP_K
    NP = P + E * BM
    nblk = NP // BM

    h2d = hidden_states.reshape(T, H)
    ids, gates, _soft, _cnt, aux = _router(h2d, gate_w)
    e1 = ids[:, 0, :].reshape(T)
    e2 = ids[:, 1, :].reshape(T)
    g1 = gates[:, 0, :].reshape(T)
    g2 = gates[:, 1, :].reshape(T)

    e_flat = jnp.stack([e1, e2], axis=1).reshape(P)
    g_flat = jnp.stack([g1, g2], axis=1).reshape(P)
    onehot = (e_flat[:, None] == jnp.arange(E)[None, :]).astype(jnp.int32)
    counts = jnp.sum(onehot, axis=0)
    nblk_e = (counts + BM - 1) // BM
    blk_start = jnp.concatenate([jnp.zeros((1,), jnp.int32), jnp.cumsum(nblk_e)[:-1]])
    astart = blk_start * BM
    u = jnp.sum(nblk_e)
    ranks = jnp.cumsum(onehot, axis=0) - 1
    rank = jnp.take_along_axis(ranks, e_flat[:, None], axis=1)[:, 0]
    pos = astart[e_flat] + rank
    tid = jnp.arange(P, dtype=jnp.int32) // TOP_K
    tid_sorted = jnp.zeros((NP,), jnp.int32).at[pos].set(tid)
    gate_sorted = jnp.zeros((NP,), jnp.float32).at[pos].set(g_flat)
    p0 = pos.reshape(T, TOP_K)[:, 0].astype(jnp.int32)
    p1 = pos.reshape(T, TOP_K)[:, 1].astype(jnp.int32)

    bidx = jnp.arange(nblk, dtype=jnp.int32)
    blk_cum = jnp.cumsum(nblk_e)
    be = jnp.searchsorted(blk_cum, bidx, side="right").astype(jnp.int32)
    last_e = jnp.take(be, u - 1)
    vd = (bidx < u).astype(jnp.int32)
    xi = jnp.where(bidx < u, bidx, u - 1).astype(jnp.int32)
    wi = jnp.where(bidx < u, be, last_e).astype(jnp.int32)

    x_pad = _make_gather(T, H, NP)(h2d, tid_sorted)

    xb = x_pad.astype(jnp.bfloat16)
    g3d = gate_sorted.reshape(nblk, 1, BM)
    w1b = w1.astype(jnp.bfloat16)
    w3b = w3.astype(jnp.bfloat16)
    w2b = w2.astype(jnp.bfloat16)
    y_pad = _gemm(xb, g3d, w1b, w3b, w2b, xi, wi, vd, NP, H, F)

    out = _make_combine(T, H, NP)(y_pad, p0, p1)
    return out.reshape(B, S, H), aux[0, 0]
